# baseline (speedup 1.0000x reference)
import functools

import jax
import jax.numpy as jnp
from jax import lax
from jax.experimental import pallas as pl
from jax.experimental.pallas import tpu as pltpu
from jax.experimental.pallas import tpu_sc as plsc

WORLD = 5
OPTS = 6
B = 16384
NC = 2
NS = 16
L = 16
NW = NC * NS
CHUNK = B // NW
TAB = 2 * WORLD * OPTS
GAMMA = 0.99
NACC = 4


def _sc_partials(tab, states, next_states, actions, rewards, dones):
    mesh = plsc.VectorSubcoreMesh(core_axis_name="c", subcore_axis_name="s")

    @functools.partial(
        pl.kernel,
        mesh=mesh,
        out_type=jax.ShapeDtypeStruct((NW // 8, 8 * L), jnp.float32),
        compiler_params=pltpu.CompilerParams(needs_layout_passes=False),
        scratch_types=[
            pltpu.VMEM((3 * CHUNK,), jnp.int32),
            pltpu.VMEM((2 * CHUNK,), jnp.float32),
            pltpu.VMEM((TAB,), jnp.float32),
            pltpu.VMEM((L,), jnp.float32),
            pltpu.SemaphoreType.DMA,
            pltpu.SemaphoreType.DMA,
        ],
    )
    def k(tab_hbm, st_hbm, ns_hbm, ac_hbm, rw_hbm, dn_hbm, out_hbm,
          i_v, f_v, tab_v, part_v, sem_a, sem_b):
        wid = lax.axis_index("s") * NC + lax.axis_index("c")
        base = wid * CHUNK
        H = CHUNK // 2
        copies_a = [
            pltpu.async_copy(tab_hbm, tab_v, sem_a),
            pltpu.async_copy(st_hbm.at[pl.ds(base, H)],
                             i_v.at[pl.ds(0, H)], sem_a),
            pltpu.async_copy(ns_hbm.at[pl.ds(base, H)],
                             i_v.at[pl.ds(CHUNK, H)], sem_a),
            pltpu.async_copy(ac_hbm.at[pl.ds(base, H)],
                             i_v.at[pl.ds(2 * CHUNK, H)], sem_a),
            pltpu.async_copy(rw_hbm.at[pl.ds(base, H)],
                             f_v.at[pl.ds(0, H)], sem_a),
            pltpu.async_copy(dn_hbm.at[pl.ds(base, H)],
                             f_v.at[pl.ds(CHUNK, H)], sem_a),
        ]
        copies_b = [
            pltpu.async_copy(st_hbm.at[pl.ds(base + H, H)],
                             i_v.at[pl.ds(H, H)], sem_b),
            pltpu.async_copy(ns_hbm.at[pl.ds(base + H, H)],
                             i_v.at[pl.ds(CHUNK + H, H)], sem_b),
            pltpu.async_copy(ac_hbm.at[pl.ds(base + H, H)],
                             i_v.at[pl.ds(2 * CHUNK + H, H)], sem_b),
            pltpu.async_copy(rw_hbm.at[pl.ds(base + H, H)],
                             f_v.at[pl.ds(H, H)], sem_b),
            pltpu.async_copy(dn_hbm.at[pl.ds(base + H, H)],
                             f_v.at[pl.ds(CHUNK + H, H)], sem_b),
        ]
        for c in copies_a:
            c.wait()

        row = jnp.minimum(lax.iota(jnp.int32, L), WORLD - 1) * OPTS
        m = plsc.load_gather(tab_v, [row + (WORLD * OPTS)])
        for j in range(1, OPTS):
            m = jnp.maximum(m, plsc.load_gather(tab_v, [row + (WORLD * OPTS + j)]))
        part_v[...] = m

        def make_body(accs_init, lo, hi):
            @plsc.parallel_loop(lo, hi, NACC * L, unroll=4, carry=accs_init)
            def accs(base_off, accs):
                accs = list(accs)
                for u in range(NACC):
                    off = base_off + u * L
                    s = i_v[pl.ds(off, L)]
                    nx = i_v[pl.ds(CHUNK + off, L)]
                    a = i_v[pl.ds(2 * CHUNK + off, L)]
                    r = f_v[pl.ds(off, L)]
                    d = f_v[pl.ds(CHUNK + off, L)]
                    mn = plsc.load_gather(part_v, [nx])
                    qsel = plsc.load_gather(tab_v, [s * OPTS + a])
                    dif = qsel - (r + (1.0 - d) * GAMMA * mn)
                    accs[u] = accs[u] + dif * dif
                return tuple(accs)
            return accs

        accs = make_body(tuple(jnp.zeros((L,), jnp.float32)
                               for _ in range(NACC)), 0, H)
        for c in copies_b:
            c.wait()
        accs = make_body(accs, H, CHUNK)

        acc = (accs[0] + accs[1]) + (accs[2] + accs[3])
        part_v[...] = acc
        pltpu.sync_copy(part_v, out_hbm.at[wid // 8, pl.ds((wid % 8) * L, L)])

    return k(tab, states, next_states, actions, rewards, dones)


def _tc_reduce(partials_2d):
    def body(p_ref, o_ref):
        o_ref[0, 0] = jnp.sum(p_ref[...]) * (1.0 / B)

    out = pl.pallas_call(
        body,
        out_shape=jax.ShapeDtypeStruct((1, 1), jnp.float32),
        out_specs=pl.BlockSpec(memory_space=pltpu.SMEM),
    )(partials_2d)
    return out[0, 0]


def kernel(qs, q_targets, states, next_states, actions, rewards, dones):
    tab = jnp.concatenate([qs.reshape(-1), q_targets.reshape(-1)])
    partials = _sc_partials(tab, states, next_states,
                            actions, rewards, dones)
    return _tc_reduce(partials)


# speedup vs baseline: 1.0109x; 1.0109x over previous
import functools

import jax
import jax.numpy as jnp
from jax import lax
from jax.experimental import pallas as pl
from jax.experimental.pallas import tpu as pltpu
from jax.experimental.pallas import tpu_sc as plsc

WORLD = 5
OPTS = 6
B = 16384
NC = 2
NS = 16
L = 16
NW = NC * NS
CHUNK = B // NW
TAB = 2 * WORLD * OPTS
GAMMA = 0.99
NACC = 4


def _sc_partials(tab, states, next_states, actions, rewards, dones):
    mesh = plsc.VectorSubcoreMesh(core_axis_name="c", subcore_axis_name="s")

    @functools.partial(
        pl.kernel,
        mesh=mesh,
        out_type=jax.ShapeDtypeStruct((NW // 8, 8 * L), jnp.float32),
        compiler_params=pltpu.CompilerParams(needs_layout_passes=False),
        scratch_types=[
            pltpu.VMEM((3 * CHUNK,), jnp.int32),
            pltpu.VMEM((2 * CHUNK,), jnp.float32),
            pltpu.VMEM((TAB,), jnp.float32),
            pltpu.VMEM((L,), jnp.float32),
            pltpu.SemaphoreType.DMA,
        ],
    )
    def k(tab_hbm, st_hbm, ns_hbm, ac_hbm, rw_hbm, dn_hbm, out_hbm,
          i_v, f_v, tab_v, part_v, sem):
        wid = lax.axis_index("s") * NC + lax.axis_index("c")
        base = wid * CHUNK
        copies = [
            pltpu.async_copy(tab_hbm, tab_v, sem),
            pltpu.async_copy(st_hbm.at[pl.ds(base, CHUNK)],
                             i_v.at[pl.ds(0, CHUNK)], sem),
            pltpu.async_copy(ns_hbm.at[pl.ds(base, CHUNK)],
                             i_v.at[pl.ds(CHUNK, CHUNK)], sem),
            pltpu.async_copy(ac_hbm.at[pl.ds(base, CHUNK)],
                             i_v.at[pl.ds(2 * CHUNK, CHUNK)], sem),
            pltpu.async_copy(rw_hbm.at[pl.ds(base, CHUNK)],
                             f_v.at[pl.ds(0, CHUNK)], sem),
            pltpu.async_copy(dn_hbm.at[pl.ds(base, CHUNK)],
                             f_v.at[pl.ds(CHUNK, CHUNK)], sem),
        ]
        for c in copies:
            c.wait()

        row = jnp.minimum(lax.iota(jnp.int32, L), WORLD - 1) * OPTS
        m = plsc.load_gather(tab_v, [row + (WORLD * OPTS)])
        for j in range(1, OPTS):
            m = jnp.maximum(m, plsc.load_gather(tab_v, [row + (WORLD * OPTS + j)]))
        part_v[...] = m

        @plsc.parallel_loop(0, CHUNK, NACC * L, unroll=4,
                            carry=tuple(jnp.zeros((L,), jnp.float32)
                                        for _ in range(NACC)))
        def accs(base_off, accs):
            accs = list(accs)
            for u in range(NACC):
                off = base_off + u * L
                s = i_v[pl.ds(off, L)]
                nx = i_v[pl.ds(CHUNK + off, L)]
                a = i_v[pl.ds(2 * CHUNK + off, L)]
                r = f_v[pl.ds(off, L)]
                d = f_v[pl.ds(CHUNK + off, L)]
                mn = plsc.load_gather(part_v, [nx])
                qsel = plsc.load_gather(tab_v, [s * OPTS + a])
                dif = qsel - (r + (1.0 - d) * GAMMA * mn)
                accs[u] = accs[u] + dif * dif
            return tuple(accs)

        acc = (accs[0] + accs[1]) + (accs[2] + accs[3])
        part_v[...] = acc
        pltpu.sync_copy(part_v, out_hbm.at[wid // 8, pl.ds((wid % 8) * L, L)])

    return k(tab, states, next_states, actions, rewards, dones)


def _tc_reduce(partials_2d):
    def body(p_ref, o_ref):
        o_ref[0, 0] = jnp.sum(p_ref[...]) * (1.0 / B)

    out = pl.pallas_call(
        body,
        out_shape=jax.ShapeDtypeStruct((1, 1), jnp.float32),
        out_specs=pl.BlockSpec(memory_space=pltpu.SMEM),
    )(partials_2d)
    return out[0, 0]


def kernel(qs, q_targets, states, next_states, actions, rewards, dones):
    tab = jnp.concatenate([qs.reshape(-1), q_targets.reshape(-1)])
    partials = _sc_partials(tab, states, next_states,
                            actions, rewards, dones)
    return _tc_reduce(partials)
